# Initial kernel scaffold; baseline (speedup 1.0000x reference)
import functools

import jax
import jax.numpy as jnp
from jax import lax
from jax.experimental import pallas as pl
from jax.experimental.pallas import tpu as pltpu
import jax.experimental.pallas.tpu_sc as plsc

N = 320000
D = 128
NUM_SEG = 16384
HALF = NUM_SEG // 2
NSUB = 16
R = 256
NWIN16 = N // 16
TRASH = HALF


def _body(x_hbm, seg_hbm, out_hbm, acc_sh, buf, idxraw, idx2, probe):
    c = lax.axis_index("c")
    s = lax.axis_index("s")

    zero16 = jnp.zeros((16,), jnp.float32)

    def zrow(r, _):
        for j in range(D // 16):
            buf[r, pl.ds(j * 16, 16)] = zero16
        return 0

    lax.fori_loop(0, R, zrow, 0)
    pltpu.sync_copy(buf, acc_sh.at[pl.ds(s * 512, R)])
    pltpu.sync_copy(buf, acc_sh.at[pl.ds(s * 512 + R, R)])
    plsc.subcore_barrier()

    def probe_cnt(w):
        pltpu.sync_copy(seg_hbm.at[pl.ds(w * 16, 16)], probe)
        m = probe[...] < HALF
        return jnp.max(plsc.all_reduce_population_count(m))

    def bstep(_, st):
        lo, hi, cnt_lo = st
        active = (hi - lo) > 1
        mid = lo + (hi - lo) // 2
        cnt = probe_cnt(jnp.maximum(mid, 0))
        take = active & (cnt > 0)
        lo2 = jnp.where(take, mid, lo)
        cnt2 = jnp.where(take, cnt, cnt_lo)
        hi2 = jnp.where(active & (cnt == 0), mid, hi)
        return lo2, hi2, cnt2

    lo, hi, cnt_lo = lax.fori_loop(
        0, 15, bstep,
        (jnp.int32(-1), jnp.int32(NWIN16), jnp.int32(0)))
    p_split = jnp.where(lo < 0, 0, lo * 16 + cnt_lo).astype(jnp.int32)

    base = jnp.where(c == 0, 0, p_split)
    limit = jnp.where(c == 0, p_split, N)
    length = limit - base
    r0 = base + (s * length) // NSUB
    r1 = base + ((s + 1) * length) // NSUB
    a0 = r0 - lax.rem(r0, 8)
    nwin = (r1 - a0 + (R - 1)) // R

    seg_base = c * HALF
    lane = lax.iota(jnp.int32, 16)

    def win(k, _):
        wlo = a0 + k * R
        st = jnp.minimum(wlo, N - R)
        lo_k = jnp.maximum(wlo, r0)
        hi_k = jnp.minimum(wlo + R, r1)
        pltpu.sync_copy(x_hbm.at[pl.ds(st, R)], buf)
        pltpu.sync_copy(seg_hbm.at[pl.ds(st, R)], idxraw)
        for j in range(R // 16):
            seg = idxraw[pl.ds(j * 16, 16)]
            g = st + j * 16 + lane
            valid = (g >= lo_k) & (g < hi_k)
            li = jnp.where(valid, seg - seg_base, TRASH)
            idx2[j // 8, pl.ds((j % 8) * 16, 16)] = li
        pltpu.sync_copy(buf.at[pl.ds(0, 128)], acc_sh.at[idx2.at[0]],
                        add=True)
        pltpu.sync_copy(buf.at[pl.ds(128, 128)], acc_sh.at[idx2.at[1]],
                        add=True)
        return 0

    lax.fori_loop(0, nwin, win, 0)
    plsc.subcore_barrier()

    pltpu.sync_copy(acc_sh.at[pl.ds(s * 512, 512)],
                    out_hbm.at[pl.ds(c * HALF + s * 512, 512)])


@jax.jit
def _segment_sum(outputs, atom_split):
    mesh = plsc.VectorSubcoreMesh(core_axis_name="c", subcore_axis_name="s")
    return pl.kernel(
        _body,
        out_type=jax.ShapeDtypeStruct((NUM_SEG, D), jnp.float32),
        mesh=mesh,
        scratch_types=[
            pltpu.MemorySpace.VMEM_SHARED((HALF + 1, D), jnp.float32),
            pltpu.VMEM((R, D), jnp.float32),
            pltpu.VMEM((R,), jnp.int32),
            pltpu.VMEM((2, 128), jnp.int32),
            pltpu.VMEM((16,), jnp.int32),
        ],
    )(outputs, atom_split)


def kernel(outputs, pair_features, atom_split, dummy):
    return (_segment_sum(outputs, atom_split), pair_features)


# speedup vs baseline: 5.1260x; 5.1260x over previous
import functools

import jax
import jax.numpy as jnp
from jax import lax
from jax.experimental import pallas as pl
from jax.experimental.pallas import tpu as pltpu
import jax.experimental.pallas.tpu_sc as plsc

N = 320000
D = 128
NUM_SEG = 16384
HALF = NUM_SEG // 2
NSUB = 16
R = 256
NWIN16 = N // 16
TRASH = HALF


def _body(x_hbm, seg_hbm, out_hbm, acc_sh, buf, idxraw, idx2, probe):
    c = lax.axis_index("c")
    s = lax.axis_index("s")

    zero16 = jnp.zeros((16,), jnp.float32)

    def zrow(r, _):
        for j in range(D // 16):
            buf[r, pl.ds(j * 16, 16)] = zero16
        return 0

    lax.fori_loop(0, R, zrow, 0)
    pltpu.sync_copy(buf, acc_sh.at[pl.ds(pl.multiple_of(s * 512, 512), R)])
    pltpu.sync_copy(buf, acc_sh.at[pl.ds(pl.multiple_of(s * 512 + R, R), R)])
    plsc.subcore_barrier()

    def probe_win(w):
        pltpu.sync_copy(seg_hbm.at[pl.ds(pl.multiple_of(w * 16, 16), 16)],
                        probe)

    def bstep(_, st):
        lo, hi = st
        active = (hi - lo) > 1
        mid = lo + (hi - lo) // 2
        probe_win(jnp.maximum(mid, 0))
        pred = probe[...][0] < HALF
        take = active & pred
        lo2 = jnp.where(take, mid, lo)
        hi2 = jnp.where(active & (~pred), mid, hi)
        return lo2, hi2

    lo, hi = lax.fori_loop(
        0, 15, bstep, (jnp.int32(-1), jnp.int32(NWIN16)))
    probe_win(jnp.maximum(lo, 0))
    pv = probe[...]
    cnt_lo = jnp.int32(0)
    for i in range(16):
        cnt_lo = cnt_lo + jnp.minimum(
            jnp.maximum(HALF - pv[i], 0), 1)
    p_split = jnp.where(lo < 0, 0, lo * 16 + cnt_lo).astype(jnp.int32)

    base = jnp.where(c == 0, 0, p_split)
    limit = jnp.where(c == 0, p_split, N)
    length = limit - base
    r0 = base + (s * length) // NSUB
    r1 = base + ((s + 1) * length) // NSUB
    a0 = r0 - lax.rem(r0, 8)
    nwin = (r1 - a0 + (R - 1)) // R

    seg_base = c * HALF
    lane = lax.iota(jnp.int32, 16)

    def win(k, _):
        wlo = a0 + k * R
        st = pl.multiple_of(jnp.minimum(wlo, N - R), 8)
        lo_k = jnp.maximum(wlo, r0)
        hi_k = jnp.minimum(wlo + R, r1)
        pltpu.sync_copy(x_hbm.at[pl.ds(st, R)], buf)
        pltpu.sync_copy(seg_hbm.at[pl.ds(st, R)], idxraw)
        for j in range(R // 16):
            seg = idxraw[pl.ds(j * 16, 16)]
            g = st + j * 16 + lane
            valid = (g >= lo_k) & (g < hi_k)
            li = jnp.where(valid, seg - seg_base, TRASH)
            idx2[j // 8, pl.ds((j % 8) * 16, 16)] = li
        pltpu.sync_copy(buf.at[pl.ds(0, 128)], acc_sh.at[idx2.at[0]],
                        add=True)
        pltpu.sync_copy(buf.at[pl.ds(128, 128)], acc_sh.at[idx2.at[1]],
                        add=True)
        return 0

    lax.fori_loop(0, nwin, win, 0)
    plsc.subcore_barrier()

    pltpu.sync_copy(acc_sh.at[pl.ds(pl.multiple_of(s * 512, 512), 512)],
                    out_hbm.at[pl.ds(pl.multiple_of(c * HALF + s * 512, 512),
                                     512)])


@jax.jit
def _segment_sum(outputs, atom_split):
    mesh = plsc.VectorSubcoreMesh(core_axis_name="c", subcore_axis_name="s")
    return pl.kernel(
        _body,
        out_type=jax.ShapeDtypeStruct((NUM_SEG, D), jnp.float32),
        mesh=mesh,
        scratch_types=[
            pltpu.MemorySpace.VMEM_SHARED((HALF + 1, D), jnp.float32),
            pltpu.VMEM((R, D), jnp.float32),
            pltpu.VMEM((R,), jnp.int32),
            pltpu.VMEM((2, 128), jnp.int32),
            pltpu.VMEM((16,), jnp.int32),
        ],
    )(outputs, atom_split)


def kernel(outputs, pair_features, atom_split, dummy):
    return (_segment_sum(outputs, atom_split), pair_features)


# speedup vs baseline: 7.1737x; 1.3995x over previous
import jax
import jax.numpy as jnp
from jax import lax
from jax.experimental import pallas as pl
from jax.experimental.pallas import tpu as pltpu
import jax.experimental.pallas.tpu_sc as plsc

N = 320000
D = 128
NUM_SEG = 16384
HALF = NUM_SEG // 2
NSUB = 16
R = 128
NSLOT = 3
NWIN16 = N // 16
TRASH = HALF


def _body(x_hbm, seg_hbm, out_hbm, acc_sh,
          buf0, buf1, buf2, ix0, ix1, ix2, probe,
          sx0, sx1, sx2, si0, si1, si2, ss0, ss1, ss2):
    c = lax.axis_index("c")
    s = lax.axis_index("s")
    slots = ((buf0, ix0, sx0, si0, ss0),
             (buf1, ix1, sx1, si1, ss1),
             (buf2, ix2, sx2, si2, ss2))

    zero16 = jnp.zeros((16,), jnp.float32)

    def zrow(r, _):
        for j in range(D // 16):
            buf0[r, pl.ds(j * 16, 16)] = zero16
        return 0

    lax.fori_loop(0, R, zrow, 0)
    for t in range(512 // R):
        pltpu.sync_copy(
            buf0, acc_sh.at[pl.ds(pl.multiple_of(s * 512 + t * R, R), R)])
    plsc.subcore_barrier()

    def probe_win(w):
        pltpu.sync_copy(seg_hbm.at[pl.ds(pl.multiple_of(w * 16, 16), 16)],
                        probe)

    def bstep(_, st):
        lo, hi = st
        active = (hi - lo) > 1
        mid = lo + (hi - lo) // 2
        probe_win(jnp.maximum(mid, 0))
        pred = probe[...][0] < HALF
        take = active & pred
        lo2 = jnp.where(take, mid, lo)
        hi2 = jnp.where(active & (~pred), mid, hi)
        return lo2, hi2

    lo, hi = lax.fori_loop(
        0, 15, bstep, (jnp.int32(-1), jnp.int32(NWIN16)))
    probe_win(jnp.maximum(lo, 0))
    pv = probe[...]
    cnt_lo = jnp.int32(0)
    for i in range(16):
        cnt_lo = cnt_lo + jnp.minimum(
            jnp.maximum(HALF - pv[i], 0), 1)
    p_split = jnp.where(lo < 0, 0, lo * 16 + cnt_lo).astype(jnp.int32)

    base = jnp.where(c == 0, 0, p_split)
    limit = jnp.where(c == 0, p_split, N)
    length = limit - base
    r0 = base + (s * length) // NSUB
    r1 = base + ((s + 1) * length) // NSUB
    a0 = r0 - lax.rem(r0, 8)
    nwin = (r1 - a0 + (R - 1)) // R

    seg_base = c * HALF
    lane = lax.iota(jnp.int32, 16)

    def st_of(k):
        return pl.multiple_of(jnp.minimum(a0 + k * R, N - R), 8)

    def issue_gather(k, slot):
        buf, ix, sx, si, _ = slot

        @pl.when(k < nwin)
        def _():
            st = st_of(k)
            pltpu.async_copy(x_hbm.at[pl.ds(st, R)], buf, sx)
            pltpu.async_copy(seg_hbm.at[pl.ds(st, R)], ix.at[0], si)

    def consume(k, slot):
        buf, ix, sx, si, ss = slot

        @pl.when(k < nwin)
        def _():
            pltpu.make_async_copy(x_hbm.at[pl.ds(0, R)], buf, sx).wait()
            pltpu.make_async_copy(seg_hbm.at[pl.ds(0, R)], ix.at[0],
                                  si).wait()
            st = st_of(k)
            lo_k = jnp.maximum(a0 + k * R, r0)
            hi_k = jnp.minimum(a0 + k * R + R, r1)
            for j in range(R // 16):
                seg = ix[0, pl.ds(j * 16, 16)]
                g = st + j * 16 + lane
                valid = (g >= lo_k) & (g < hi_k)
                li = jnp.where(valid, seg - seg_base, TRASH)
                ix[0, pl.ds(j * 16, 16)] = li
            pltpu.async_copy(buf, acc_sh.at[ix.at[0]], ss, add=True)

    def wait_scatter(k, slot):
        buf, ix, _, _, ss = slot

        @pl.when((k >= 0) & (k < nwin))
        def _():
            pltpu.make_async_copy(buf, acc_sh.at[ix.at[0]], ss).wait()

    issue_gather(jnp.int32(0), slots[0])
    issue_gather(jnp.int32(1), slots[1])

    def outer(g, _):
        for b in range(NSLOT):
            k = g * NSLOT + b
            consume(k, slots[b])
            pb = (b - 1) % NSLOT
            wait_scatter(k - 1, slots[pb])
            issue_gather(k + 2, slots[pb])
        return 0

    lax.fori_loop(0, (nwin + NSLOT - 1) // NSLOT, outer, 0)
    @pl.when((lax.rem(nwin, NSLOT) == 0) & (nwin > 0))
    def _():
        buf, ix, _, _, ss = slots[NSLOT - 1]
        pltpu.make_async_copy(buf, acc_sh.at[ix.at[0]], ss).wait()

    plsc.subcore_barrier()

    pltpu.sync_copy(acc_sh.at[pl.ds(pl.multiple_of(s * 512, 512), 512)],
                    out_hbm.at[pl.ds(pl.multiple_of(c * HALF + s * 512, 512),
                                     512)])


@jax.jit
def _segment_sum(outputs, atom_split):
    mesh = plsc.VectorSubcoreMesh(core_axis_name="c", subcore_axis_name="s")
    return pl.kernel(
        _body,
        out_type=jax.ShapeDtypeStruct((NUM_SEG, D), jnp.float32),
        mesh=mesh,
        scratch_types=[
            pltpu.MemorySpace.VMEM_SHARED((HALF + 1, D), jnp.float32),
            pltpu.VMEM((R, D), jnp.float32),
            pltpu.VMEM((R, D), jnp.float32),
            pltpu.VMEM((R, D), jnp.float32),
            pltpu.VMEM((1, R), jnp.int32),
            pltpu.VMEM((1, R), jnp.int32),
            pltpu.VMEM((1, R), jnp.int32),
            pltpu.VMEM((16,), jnp.int32),
            pltpu.SemaphoreType.DMA,
            pltpu.SemaphoreType.DMA,
            pltpu.SemaphoreType.DMA,
            pltpu.SemaphoreType.DMA,
            pltpu.SemaphoreType.DMA,
            pltpu.SemaphoreType.DMA,
            pltpu.SemaphoreType.DMA,
            pltpu.SemaphoreType.DMA,
            pltpu.SemaphoreType.DMA,
        ],
    )(outputs, atom_split)


def kernel(outputs, pair_features, atom_split, dummy):
    return (_segment_sum(outputs, atom_split), pair_features)
